# baseline (speedup 1.0000x reference)
import functools

import jax
import jax.numpy as jnp
from jax import lax
from jax.experimental import pallas as pl
from jax.experimental.pallas import tpu as pltpu
from jax.experimental.pallas import tpu_sc as plsc

B = 4096
SEQ = 200
D = 128
LANES = 16
DCHUNKS = D // LANES
S0 = 104
S1 = SEQ - S0

NC = 2
NS = 16
NW = NC * NS
RPW = B // NW
NBUF = 2
UNROLL = 4


def _build():
    mesh = plsc.VectorSubcoreMesh(core_axis_name="c", subcore_axis_name="s")

    @functools.partial(
        pl.kernel,
        mesh=mesh,
        out_type=jax.ShapeDtypeStruct((B, D), jnp.float32),
        scratch_types=[
            pltpu.VMEM((RPW * SEQ,), jnp.int32),
            pltpu.VMEM((SEQ, D), jnp.float32),
            pltpu.VMEM((SEQ, D), jnp.float32),
            pltpu.VMEM((SEQ, D), jnp.float32),
            pltpu.VMEM((RPW, D), jnp.float32),
            pltpu.SemaphoreType.DMA,
            pltpu.SemaphoreType.DMA,
            pltpu.SemaphoreType.DMA,
        ],
    )
    def k(x_hbm, table_hbm, out_hbm, idx_v, buf0, buf1, buf2, out_v,
          sem0, sem1, sem2):
        cid = lax.axis_index("c")
        sid = lax.axis_index("s")
        wid = sid * NC + cid
        base = wid * RPW

        pltpu.sync_copy(x_hbm.at[pl.ds(base * SEQ, RPW * SEQ)], idx_v)

        bufs = (buf0, buf1, buf2)
        sems = (sem0, sem1, sem2)

        def copies(r, j):
            return (
                pltpu.make_async_copy(
                    table_hbm.at[idx_v.at[pl.ds(r * SEQ, S0)]],
                    bufs[j].at[pl.ds(0, S0)],
                    sems[j],
                ),
                pltpu.make_async_copy(
                    table_hbm.at[idx_v.at[pl.ds(r * SEQ + S0, S1)]],
                    bufs[j].at[pl.ds(S0, S1)],
                    sems[j],
                ),
            )

        def start(r, j):
            for c in copies(r, j):
                c.start()

        def wait(r, j):
            pltpu.make_async_copy(
                table_hbm.at[pl.ds(0, SEQ)], bufs[j], sems[j]
            ).wait()

        def reduce_row(r, j):
            buf = bufs[j]

            def body(i, accs):
                return tuple(
                    accs[d] + buf[i, pl.ds(d * LANES, LANES)]
                    for d in range(DCHUNKS)
                )

            accs = lax.fori_loop(
                0, SEQ, body,
                tuple(jnp.zeros((LANES,), jnp.float32) for _ in range(DCHUNKS)),
                unroll=UNROLL,
            )
            for d in range(DCHUNKS):
                out_v[r, pl.ds(d * LANES, LANES)] = accs[d] * (1.0 / SEQ)

        for j in range(NBUF):
            start(j, j)

        n_main = (RPW - NBUF) // NBUF
        def outer(i, carry):
            for j in range(NBUF):
                r = NBUF * i + j
                wait(r, j)
                reduce_row(r, j)
                start(r + NBUF, j)
            return carry

        lax.fori_loop(0, n_main, outer, 0)

        for r in range(NBUF * n_main, RPW):
            j = r % NBUF
            wait(r, j)
            reduce_row(r, j)
            if r + NBUF < RPW:
                start(r + NBUF, j)

        pltpu.sync_copy(out_v, out_hbm.at[pl.ds(base, RPW)])

    return k


_pooled_lookup = _build()


def kernel(x, table):
    return _pooled_lookup(x.astype(jnp.int32).reshape(-1), table)


# speedup vs baseline: 1.2080x; 1.2080x over previous
import functools

import jax
import jax.numpy as jnp
from jax import lax
from jax.experimental import pallas as pl
from jax.experimental.pallas import tpu as pltpu
from jax.experimental.pallas import tpu_sc as plsc

B = 4096
SEQ = 200
D = 128
LANES = 16
DCHUNKS = D // LANES
S0 = 104
S1 = SEQ - S0
SLIV = 208

NC = 2
NS = 16
NW = NC * NS
RPW = B // NW
QROWS = 32
NBUF = 4
UNROLL = 4


def _build():
    mesh = plsc.VectorSubcoreMesh(core_axis_name="c", subcore_axis_name="s")

    @functools.partial(
        pl.kernel,
        mesh=mesh,
        out_type=jax.ShapeDtypeStruct((B, D), jnp.float32),
        scratch_types=[
            pltpu.VMEM((QROWS * SEQ + 16,), jnp.int32),
            pltpu.VMEM((SLIV,), jnp.int32),
            pltpu.VMEM((SLIV,), jnp.int32),
            pltpu.VMEM((SLIV,), jnp.int32),
            pltpu.VMEM((SLIV,), jnp.int32),
            pltpu.VMEM((SEQ, D), jnp.float32),
            pltpu.VMEM((SEQ, D), jnp.float32),
            pltpu.VMEM((SEQ, D), jnp.float32),
            pltpu.VMEM((SEQ, D), jnp.float32),
            pltpu.VMEM((RPW, D), jnp.float32),
            pltpu.SemaphoreType.DMA,
            pltpu.SemaphoreType.DMA,
            pltpu.SemaphoreType.DMA,
            pltpu.SemaphoreType.DMA,
        ],
    )
    def k(x_hbm, table_hbm, out_hbm, idx_q, iv0, iv1, iv2, iv3,
          buf0, buf1, buf2, buf3, out_v, sem0, sem1, sem2, sem3):
        cid = lax.axis_index("c")
        sid = lax.axis_index("s")
        wid = sid * NC + cid
        base = wid * RPW

        def stage_quarter(first_row):
            pltpu.sync_copy(
                x_hbm.at[pl.ds((base + first_row) * SEQ, QROWS * SEQ)],
                idx_q.at[pl.ds(0, QROWS * SEQ)],
            )

        stage_quarter(0)

        ivs = (iv0, iv1, iv2, iv3)
        bufs = (buf0, buf1, buf2, buf3)
        sems = (sem0, sem1, sem2, sem3)

        def start(r, j):
            roff = lax.rem(r, QROWS) * SEQ
            iv = ivs[j]
            for c in range(SLIV // LANES):
                iv[pl.ds(c * LANES, LANES)] = idx_q[
                    pl.ds(roff + c * LANES, LANES)]
            pltpu.async_copy(
                table_hbm.at[iv.at[pl.ds(0, S0)]],
                bufs[j].at[pl.ds(0, S0)],
                sems[j],
            )
            pltpu.async_copy(
                table_hbm.at[iv.at[pl.ds(S0, S1)]],
                bufs[j].at[pl.ds(S0, S1)],
                sems[j],
            )

        def wait(j):
            pltpu.make_async_copy(
                table_hbm.at[pl.ds(0, SEQ)], bufs[j], sems[j]
            ).wait()

        def reduce_row(r, j):
            buf = bufs[j]

            def body(i, accs):
                return tuple(
                    accs[d] + buf[i, pl.ds(d * LANES, LANES)]
                    for d in range(DCHUNKS)
                )

            accs = lax.fori_loop(
                0, SEQ, body,
                tuple(jnp.zeros((LANES,), jnp.float32) for _ in range(DCHUNKS)),
                unroll=UNROLL,
            )
            for d in range(DCHUNKS):
                out_v[r, pl.ds(d * LANES, LANES)] = accs[d] * (1.0 / SEQ)

        for j in range(NBUF):
            start(j, j)

        n_main = (RPW - NBUF) // NBUF
        def outer(i, carry):
            @pl.when(lax.rem(i, QROWS // NBUF) == QROWS // NBUF - 1)
            def _():
                stage_quarter(NBUF * (i + 1))

            for j in range(NBUF):
                r = NBUF * i + j
                wait(j)
                reduce_row(r, j)
                start(r + NBUF, j)
            return carry

        lax.fori_loop(0, n_main, outer, 0)

        for r in range(NBUF * n_main, RPW):
            j = r % NBUF
            wait(j)
            reduce_row(r, j)

        pltpu.sync_copy(out_v, out_hbm.at[pl.ds(base, RPW)])

    return k


_pooled_lookup = _build()


def kernel(x, table):
    return _pooled_lookup(x.astype(jnp.int32).reshape(-1), table)


# speedup vs baseline: 1.2273x; 1.0159x over previous
import functools

import jax
import jax.numpy as jnp
from jax import lax
from jax.experimental import pallas as pl
from jax.experimental.pallas import tpu as pltpu
from jax.experimental.pallas import tpu_sc as plsc

B = 4096
SEQ = 200
D = 128
LANES = 16
DCHUNKS = D // LANES
S0 = 104
S1 = SEQ - S0

NC = 2
NS = 16
NW = NC * NS
RPW = B // NW
NBUF = 3
UNROLL = 4


def _build():
    mesh = plsc.VectorSubcoreMesh(core_axis_name="c", subcore_axis_name="s")

    @functools.partial(
        pl.kernel,
        mesh=mesh,
        out_type=jax.ShapeDtypeStruct((B, D), jnp.float32),
        scratch_types=[
            pltpu.VMEM((RPW * SEQ,), jnp.int32),
            pltpu.VMEM((SEQ, D), jnp.float32),
            pltpu.VMEM((SEQ, D), jnp.float32),
            pltpu.VMEM((SEQ, D), jnp.float32),
            pltpu.VMEM((RPW, D), jnp.float32),
            pltpu.SemaphoreType.DMA,
            pltpu.SemaphoreType.DMA,
            pltpu.SemaphoreType.DMA,
        ],
    )
    def k(x_hbm, table_hbm, out_hbm, idx_v, buf0, buf1, buf2, out_v,
          sem0, sem1, sem2):
        cid = lax.axis_index("c")
        sid = lax.axis_index("s")
        wid = sid * NC + cid
        base = wid * RPW

        pltpu.sync_copy(x_hbm.at[pl.ds(base * SEQ, RPW * SEQ)], idx_v)

        bufs = (buf0, buf1, buf2)
        sems = (sem0, sem1, sem2)

        def start(r, j):
            pltpu.async_copy(
                table_hbm.at[idx_v.at[pl.ds(r * SEQ, S0)]],
                bufs[j].at[pl.ds(0, S0)],
                sems[j],
            )
            pltpu.async_copy(
                table_hbm.at[idx_v.at[pl.ds(r * SEQ + S0, S1)]],
                bufs[j].at[pl.ds(S0, S1)],
                sems[j],
            )

        def wait(j):
            pltpu.make_async_copy(
                table_hbm.at[pl.ds(0, SEQ)], bufs[j], sems[j]
            ).wait()

        def reduce_row(r, j):
            buf = bufs[j]

            def body(i, accs):
                return tuple(
                    accs[d] + buf[i, pl.ds(d * LANES, LANES)]
                    for d in range(DCHUNKS)
                )

            accs = lax.fori_loop(
                0, SEQ, body,
                tuple(jnp.zeros((LANES,), jnp.float32) for _ in range(DCHUNKS)),
                unroll=UNROLL,
            )
            for d in range(DCHUNKS):
                out_v[r, pl.ds(d * LANES, LANES)] = accs[d] * (1.0 / SEQ)

        for j in range(NBUF):
            start(j, j)

        n_main = (RPW - NBUF) // NBUF
        def outer(i, carry):
            for j in range(NBUF):
                r = NBUF * i + j
                wait(j)
                reduce_row(r, j)
                start(r + NBUF, j)
            return carry

        lax.fori_loop(0, n_main, outer, 0)

        for r in range(NBUF * n_main, RPW):
            j = r % NBUF
            wait(j)
            reduce_row(r, j)
            if r + NBUF < RPW:
                start(r + NBUF, j)

        pltpu.sync_copy(out_v, out_hbm.at[pl.ds(base, RPW)])

    return k


_pooled_lookup = _build()


def kernel(x, table):
    return _pooled_lookup(x.astype(jnp.int32).reshape(-1), table)


# speedup vs baseline: 1.2311x; 1.0031x over previous
import functools

import jax
import jax.numpy as jnp
from jax import lax
from jax.experimental import pallas as pl
from jax.experimental.pallas import tpu as pltpu
from jax.experimental.pallas import tpu_sc as plsc

B = 4096
SEQ = 200
D = 128
LANES = 16
DCHUNKS = D // LANES
S0 = 104
S1 = SEQ - S0

NC = 2
NS = 16
NW = NC * NS
RPW = B // NW
NBUF = 3
UNROLL = 4


def _build():
    mesh = plsc.VectorSubcoreMesh(core_axis_name="c", subcore_axis_name="s")

    @functools.partial(
        pl.kernel,
        mesh=mesh,
        out_type=jax.ShapeDtypeStruct((B, D), jnp.float32),
        scratch_types=[
            pltpu.VMEM((RPW * SEQ,), jnp.int32),
            pltpu.VMEM((SEQ, D), jnp.float32),
            pltpu.VMEM((SEQ, D), jnp.float32),
            pltpu.VMEM((SEQ, D), jnp.float32),
            pltpu.VMEM((RPW, D), jnp.float32),
            pltpu.SemaphoreType.DMA,
            pltpu.SemaphoreType.DMA,
            pltpu.SemaphoreType.DMA,
        ],
    )
    def k(x_hbm, table_hbm, out_hbm, idx_v, buf0, buf1, buf2, out_v,
          sem0, sem1, sem2):
        cid = lax.axis_index("c")
        sid = lax.axis_index("s")
        wid = sid * NC + cid
        base = wid * RPW

        pltpu.sync_copy(x_hbm.at[pl.ds(base * SEQ, RPW * SEQ)], idx_v)

        bufs = (buf0, buf1, buf2)
        sems = (sem0, sem1, sem2)

        def start(r, j):
            for (o, n) in ((0, 56), (56, 48), (104, 48), (152, 48)):
                pltpu.async_copy(
                    table_hbm.at[idx_v.at[pl.ds(r * SEQ + o, n)]],
                    bufs[j].at[pl.ds(o, n)],
                    sems[j],
                )

        def wait(j):
            pltpu.make_async_copy(
                table_hbm.at[pl.ds(0, SEQ)], bufs[j], sems[j]
            ).wait()

        def reduce_row(r, j):
            buf = bufs[j]

            def body(i, accs):
                return tuple(
                    accs[d] + buf[i, pl.ds(d * LANES, LANES)]
                    for d in range(DCHUNKS)
                )

            accs = lax.fori_loop(
                0, SEQ, body,
                tuple(jnp.zeros((LANES,), jnp.float32) for _ in range(DCHUNKS)),
                unroll=UNROLL,
            )
            for d in range(DCHUNKS):
                out_v[r, pl.ds(d * LANES, LANES)] = accs[d] * (1.0 / SEQ)

        for j in range(NBUF):
            start(j, j)

        n_main = (RPW - NBUF) // NBUF
        def outer(i, carry):
            for j in range(NBUF):
                r = NBUF * i + j
                wait(j)
                reduce_row(r, j)
                start(r + NBUF, j)
            return carry

        lax.fori_loop(0, n_main, outer, 0)

        for r in range(NBUF * n_main, RPW):
            j = r % NBUF
            wait(j)
            reduce_row(r, j)
            if r + NBUF < RPW:
                start(r + NBUF, j)

        pltpu.sync_copy(out_v, out_hbm.at[pl.ds(base, RPW)])

    return k


_pooled_lookup = _build()


def kernel(x, table):
    return _pooled_lookup(x.astype(jnp.int32).reshape(-1), table)
